# baseline (speedup 1.0000x reference)
import jax
import jax.numpy as jnp
from jax import lax
from jax.experimental import pallas as pl
from jax.experimental.pallas import tpu as pltpu

_VMEM_LIMIT = 100 * 1024 * 1024
_EPS = 1e-6


def _stem_kernel(x_ref, w_ref, b_ref, o_ref, st_ref):
    acc = jnp.dot(x_ref[0], w_ref[...],
                  preferred_element_type=jnp.float32) + b_ref[...]
    o_ref[0] = acc.astype(o_ref.dtype)
    s = jnp.sum(acc, axis=0)
    ss = jnp.sum(acc * acc, axis=0)
    st_ref[0] = jnp.concatenate([s[None, :], ss[None, :]], axis=0)


def _scale_shift(s, ss, gm_ref, g_ref, bt_ref, inv):
    mean = jnp.dot(s, gm_ref[...], preferred_element_type=jnp.float32,
                   precision=lax.Precision.HIGHEST) * inv
    ex2 = jnp.dot(ss, gm_ref[...], preferred_element_type=jnp.float32,
                  precision=lax.Precision.HIGHEST) * inv
    var = ex2 - mean * mean
    scale = g_ref[...] * lax.rsqrt(var + _EPS)
    shift = bt_ref[...] - mean * scale
    return scale, shift


def _conv_chunks(load, scale, shift, w_ref, cb_ref, S, cin, rt):
    for r0 in range(0, S, rt):
        lo = max(r0 - 1, 0)
        hi = min(r0 + rt + 1, S)
        y = load(lo * S, hi * S) * scale + shift
        y = y * jax.nn.sigmoid(y)
        yb = y.astype(jnp.bfloat16).reshape(hi - lo, S, cin)
        sl = jnp.pad(yb, ((1 - (r0 - lo), 1 - (hi - r0 - rt)),
                          (1, 1), (0, 0)))
        patches = jnp.concatenate(
            [sl[dy:dy + rt, dx:dx + S, :].reshape(rt * S, cin)
             for dy in range(3) for dx in range(3)], axis=-1)
        yield r0, (jnp.dot(patches, w_ref[...],
                           preferred_element_type=jnp.float32) + cb_ref[...])


def _make_resblock_kernel(S, cin, cout, cg1, cg2, has_proj, do_pool):
    hw = S * S
    inv1 = 1.0 / float(hw * cg1)
    inv2 = 1.0 / float(hw * cg2)

    def body(*refs):
        (x_ref, stin_ref, g1_ref, b1_ref, gm1_ref, w1_ref, cb1_ref,
         g2_ref, b2_ref, gm2_ref, w2_ref, cb2_ref) = refs[:12]
        n = 12
        scw_ref = scb_ref = None
        if has_proj:
            scw_ref, scb_ref = refs[12:14]
            n = 14
        o_ref = refs[n]
        pool_ref = stp_ref = None
        if do_pool:
            pool_ref, stp_ref = refs[n + 1], refs[n + 2]
            h_s, st2_s = refs[n + 3], refs[n + 4]
        else:
            h_s, st2_s = refs[n + 1], refs[n + 2]

        p = pl.program_id(1)

        rt = max(2, min(8, 512 // S, S))

        @pl.when(p == 0)
        def _phase_conv1():
            scale, shift = _scale_shift(stin_ref[0, 0:1, :], stin_ref[0, 1:2, :],
                                        gm1_ref, g1_ref, b1_ref, inv1)
            ts = tss = 0.0
            for r0, acc in _conv_chunks(
                    lambda a, b: x_ref[0, a:b, :].astype(jnp.float32),
                    scale, shift, w1_ref, cb1_ref, S, cin, rt):
                h_s[r0 * S:(r0 + rt) * S, :] = acc.astype(h_s.dtype)
                ts = ts + jnp.sum(acc, axis=0)
                tss = tss + jnp.sum(acc * acc, axis=0)
            st2_s[...] = jnp.concatenate([ts[None, :], tss[None, :]], axis=0)

        @pl.when(p == 1)
        def _phase_conv2():
            scale, shift = _scale_shift(st2_s[0:1, :], st2_s[1:2, :],
                                        gm2_ref, g2_ref, b2_ref, inv2)
            ps = pss = 0.0
            for r0, acc in _conv_chunks(
                    lambda a, b: h_s[a:b, :].astype(jnp.float32),
                    scale, shift, w2_ref, cb2_ref, S, cout, rt):
                a, b = r0 * S, (r0 + rt) * S
                if has_proj:
                    acc = acc + (jnp.dot(x_ref[0, a:b, :], scw_ref[...],
                                         preferred_element_type=jnp.float32)
                                 + scb_ref[...])
                else:
                    acc = acc + x_ref[0, a:b, :].astype(jnp.float32)
                o_ref[0, a:b, :] = acc.astype(o_ref.dtype)
                if do_pool:
                    v = acc.reshape(rt // 2, 2, S // 2, 2, cout)
                    pq = 0.25 * (v[:, 0, :, 0, :] + v[:, 0, :, 1, :]
                                 + v[:, 1, :, 0, :] + v[:, 1, :, 1, :])
                    pf = pq.reshape(rt * S // 4, cout)
                    pool_ref[0, (r0 // 2) * (S // 2):
                             (r0 // 2 + rt // 2) * (S // 2), :] = (
                        pf.astype(pool_ref.dtype))
                    ps = ps + jnp.sum(pf, axis=0)
                    pss = pss + jnp.sum(pf * pf, axis=0)
            if do_pool:
                stp_ref[0] = jnp.concatenate([ps[None, :], pss[None, :]],
                                             axis=0)

    return body


def _resblock(xf, stats, gn1, gm1, w1, cb1, gn2, gm2, w2, cb2, *, S, groups,
              sc=None, do_pool=False):
    B, hw, cin = xf.shape
    cout = w1.shape[-1]
    f32 = jnp.float32

    def _c(i):
        return lambda b, p: (b,) + (0,) * i

    in_specs = [
        pl.BlockSpec((1, hw, cin), lambda b, p: (b, 0, 0)),
        pl.BlockSpec((1, 2, cin), lambda b, p: (b, 0, 0)),
        pl.BlockSpec((1, cin), lambda b, p: (0, 0)),
        pl.BlockSpec((1, cin), lambda b, p: (0, 0)),
        pl.BlockSpec((cin, cin), lambda b, p: (0, 0)),
        pl.BlockSpec((9 * cin, cout), lambda b, p: (0, 0)),
        pl.BlockSpec((1, cout), lambda b, p: (0, 0)),
        pl.BlockSpec((1, cout), lambda b, p: (0, 0)),
        pl.BlockSpec((1, cout), lambda b, p: (0, 0)),
        pl.BlockSpec((cout, cout), lambda b, p: (0, 0)),
        pl.BlockSpec((9 * cout, cout), lambda b, p: (0, 0)),
        pl.BlockSpec((1, cout), lambda b, p: (0, 0)),
    ]
    args = [xf, stats,
            gn1[0].reshape(1, cin).astype(f32), gn1[1].reshape(1, cin).astype(f32),
            gm1, w1, cb1.reshape(1, cout).astype(f32),
            gn2[0].reshape(1, cout).astype(f32), gn2[1].reshape(1, cout).astype(f32),
            gm2, w2, cb2.reshape(1, cout).astype(f32)]
    if sc is not None:
        in_specs += [pl.BlockSpec((cin, cout), lambda b, p: (0, 0)),
                     pl.BlockSpec((1, cout), lambda b, p: (0, 0))]
        args += [sc[0].astype(jnp.bfloat16), sc[1].reshape(1, cout).astype(f32)]

    out_shapes = [jax.ShapeDtypeStruct((B, hw, cout), f32)]
    out_specs = [pl.BlockSpec((1, hw, cout), lambda b, p: (b, 0, 0))]
    if do_pool:
        out_shapes += [jax.ShapeDtypeStruct((B, hw // 4, cout), jnp.bfloat16),
                       jax.ShapeDtypeStruct((B, 2, cout), f32)]
        out_specs += [pl.BlockSpec((1, hw // 4, cout), lambda b, p: (b, 0, 0)),
                      pl.BlockSpec((1, 2, cout), lambda b, p: (b, 0, 0))]

    return pl.pallas_call(
        _make_resblock_kernel(S, cin, cout, cin // groups, cout // groups,
                              sc is not None, do_pool),
        out_shape=tuple(out_shapes),
        grid=(B, 2),
        in_specs=in_specs,
        out_specs=tuple(out_specs),
        scratch_shapes=[pltpu.VMEM((hw, cout), jnp.bfloat16),
                        pltpu.VMEM((2, cout), f32)],
        compiler_params=pltpu.CompilerParams(
            dimension_semantics=("parallel", "arbitrary"),
            vmem_limit_bytes=_VMEM_LIMIT),
    )(*args)


def _group_mat(c, groups):
    gidx = jnp.arange(c) // (c // groups)
    return (gidx[:, None] == gidx[None, :]).astype(jnp.float32)


def kernel(x, conv_in_w, conv_in_b,
           r0_gn1_gamma, r0_gn1_beta, r0_conv1_w, r0_conv1_b,
           r0_gn2_gamma, r0_gn2_beta, r0_conv2_w, r0_conv2_b,
           r1_gn1_gamma, r1_gn1_beta, r1_conv1_w, r1_conv1_b,
           r1_gn2_gamma, r1_gn2_beta, r1_conv2_w, r1_conv2_b,
           r1_sc_w, r1_sc_b,
           r2_gn1_gamma, r2_gn1_beta, r2_conv1_w, r2_conv1_b,
           r2_gn2_gamma, r2_gn2_beta, r2_conv2_w, r2_conv2_b,
           r2_sc_w, r2_sc_b):
    groups = 32
    B, c0, hr, wr = x.shape
    H, W = hr // 2, wr // 2
    cu = c0 * 4
    xu = (x.reshape(B, c0, H, 2, W, 2).transpose(0, 2, 4, 1, 3, 5)
          .reshape(B, H * W, cu).astype(jnp.bfloat16))

    cin0 = conv_in_w.shape[1]
    stem_out, st = pl.pallas_call(
        _stem_kernel,
        out_shape=(jax.ShapeDtypeStruct((B, H * W, cin0), jnp.bfloat16),
                   jax.ShapeDtypeStruct((B, 2, cin0), jnp.float32)),
        grid=(B,),
        in_specs=[pl.BlockSpec((1, H * W, cu), lambda b: (b, 0, 0)),
                  pl.BlockSpec((cu, cin0), lambda b: (0, 0)),
                  pl.BlockSpec((1, cin0), lambda b: (0, 0))],
        out_specs=(pl.BlockSpec((1, H * W, cin0), lambda b: (b, 0, 0)),
                   pl.BlockSpec((1, 2, cin0), lambda b: (b, 0, 0))),
        compiler_params=pltpu.CompilerParams(
            dimension_semantics=("parallel",),
            vmem_limit_bytes=_VMEM_LIMIT),
    )(xu, conv_in_w.astype(jnp.bfloat16),
      conv_in_b.reshape(1, cin0).astype(jnp.float32))

    blocks = [
        dict(gn1=(r0_gn1_gamma, r0_gn1_beta), w1=r0_conv1_w, b1=r0_conv1_b,
             gn2=(r0_gn2_gamma, r0_gn2_beta), w2=r0_conv2_w, b2=r0_conv2_b,
             sc=None),
        dict(gn1=(r1_gn1_gamma, r1_gn1_beta), w1=r1_conv1_w, b1=r1_conv1_b,
             gn2=(r1_gn2_gamma, r1_gn2_beta), w2=r1_conv2_w, b2=r1_conv2_b,
             sc=(r1_sc_w, r1_sc_b)),
        dict(gn1=(r2_gn1_gamma, r2_gn1_beta), w1=r2_conv1_w, b1=r2_conv1_b,
             gn2=(r2_gn2_gamma, r2_gn2_beta), w2=r2_conv2_w, b2=r2_conv2_b,
             sc=(r2_sc_w, r2_sc_b)),
    ]

    feats = []
    cur, cur_st = stem_out, st
    S = H
    gmats = {}
    for i, bp in enumerate(blocks):
        cin = bp["w1"].shape[2]
        cout = bp["w1"].shape[3]
        for c in (cin, cout):
            if c not in gmats:
                gmats[c] = _group_mat(c, groups)
        w1 = bp["w1"].reshape(9 * cin, cout).astype(jnp.bfloat16)
        w2 = bp["w2"].reshape(9 * cout, cout).astype(jnp.bfloat16)
        last = i == len(blocks) - 1
        out = _resblock(
            cur, cur_st, bp["gn1"], gmats[cin], w1, bp["b1"],
            bp["gn2"], gmats[cout], w2, bp["b2"],
            S=S, groups=groups, sc=bp["sc"], do_pool=not last)
        if last:
            feat = out[0]
        else:
            feat, cur, cur_st = out
        feats.append(feat.reshape(B, S, S, cout).transpose(0, 3, 1, 2))
        S //= 2
    return feats


# speedup vs baseline: 1.0161x; 1.0161x over previous
import jax
import jax.numpy as jnp
from jax import lax
from jax.experimental import pallas as pl
from jax.experimental.pallas import tpu as pltpu

_VMEM_LIMIT = 100 * 1024 * 1024
_EPS = 1e-6


def _scale_shift(s, ss, gm_ref, g_ref, bt_ref, inv):
    mean = jnp.dot(s, gm_ref[...], preferred_element_type=jnp.float32,
                   precision=lax.Precision.HIGHEST) * inv
    ex2 = jnp.dot(ss, gm_ref[...], preferred_element_type=jnp.float32,
                  precision=lax.Precision.HIGHEST) * inv
    var = ex2 - mean * mean
    scale = g_ref[...] * lax.rsqrt(var + _EPS)
    shift = bt_ref[...] - mean * scale
    return scale, shift


def _conv_chunks(load, scale, shift, w_ref, cb_ref, S, cin, rt):
    for r0 in range(0, S, rt):
        lo = max(r0 - 1, 0)
        hi = min(r0 + rt + 1, S)
        y = load(lo * S, hi * S) * scale + shift
        y = y * jax.nn.sigmoid(y)
        yb = y.astype(jnp.bfloat16).reshape(hi - lo, S, cin)
        sl = jnp.pad(yb, ((1 - (r0 - lo), 1 - (hi - r0 - rt)),
                          (1, 1), (0, 0)))
        patches = jnp.concatenate(
            [sl[dy:dy + rt, dx:dx + S, :].reshape(rt * S, cin)
             for dy in range(3) for dx in range(3)], axis=-1)
        yield r0, (jnp.dot(patches, w_ref[...],
                           preferred_element_type=jnp.float32) + cb_ref[...])


def _make_net_kernel(S0, meta):
    def body(*refs):
        xu_ref, wst_ref, bst_ref, gm_a, gm_b, gm_c = refs[:6]
        gms = {}
        for r in (gm_a, gm_b, gm_c):
            gms[r.shape[0]] = r
        k = 6
        bparams = []
        for (S, cin, cout, cg1, cg2, has_proj, do_pool) in meta:
            nper = 8 + (2 if has_proj else 0)
            bparams.append(refs[k:k + nper])
            k += nper
        f_refs = refs[k:k + 3]
        x0_s, h0_s, p0_s, h1_s, p1_s, h2_s = refs[k + 3:k + 9]
        h_scr = [h0_s, h1_s, h2_s]
        in_scr = [x0_s, p0_s, p1_s]

        acc = jnp.dot(xu_ref[0], wst_ref[...],
                      preferred_element_type=jnp.float32) + bst_ref[...]
        x0_s[...] = acc.astype(x0_s.dtype)
        s = jnp.sum(acc, axis=0, keepdims=True)
        ss = jnp.sum(acc * acc, axis=0, keepdims=True)

        for i, (S, cin, cout, cg1, cg2, has_proj, do_pool) in enumerate(meta):
            prm = bparams[i]
            if has_proj:
                (g1, b1, w1, cb1, g2, b2, w2, cb2, scw, scb) = prm
            else:
                (g1, b1, w1, cb1, g2, b2, w2, cb2) = prm
                scw = scb = None
            x_s = in_scr[i]
            h_s = h_scr[i]
            rt = max(2, min(8, 512 // S, S))
            hw = S * S

            scale, shift = _scale_shift(s, ss, gms[cin], g1, b1,
                                        1.0 / float(hw * cg1))
            ts = tss = 0.0
            for r0, a1 in _conv_chunks(
                    lambda a, b: x_s[a:b, :].astype(jnp.float32),
                    scale, shift, w1, cb1, S, cin, rt):
                h_s[r0 * S:(r0 + rt) * S, :] = a1.astype(h_s.dtype)
                ts = ts + jnp.sum(a1, axis=0, keepdims=True)
                tss = tss + jnp.sum(a1 * a1, axis=0, keepdims=True)

            scale, shift = _scale_shift(ts, tss, gms[cout], g2, b2,
                                        1.0 / float(hw * cg2))
            ps = pss = 0.0
            for r0, a2 in _conv_chunks(
                    lambda a, b: h_s[a:b, :].astype(jnp.float32),
                    scale, shift, w2, cb2, S, cout, rt):
                a, b = r0 * S, (r0 + rt) * S
                if has_proj:
                    a2 = a2 + (jnp.dot(x_s[a:b, :], scw[...],
                                       preferred_element_type=jnp.float32)
                               + scb[...])
                else:
                    a2 = a2 + x_s[a:b, :].astype(jnp.float32)
                f_refs[i][0, a:b, :] = a2.astype(f_refs[i].dtype)
                if do_pool:
                    v = a2.reshape(rt // 2, 2, S // 2, 2, cout)
                    pq = 0.25 * (v[:, 0, :, 0, :] + v[:, 0, :, 1, :]
                                 + v[:, 1, :, 0, :] + v[:, 1, :, 1, :])
                    pf = pq.reshape(rt * S // 4, cout)
                    in_scr[i + 1][(r0 // 2) * (S // 2):
                                  (r0 // 2 + rt // 2) * (S // 2), :] = (
                        pf.astype(in_scr[i + 1].dtype))
                    ps = ps + jnp.sum(pf, axis=0, keepdims=True)
                    pss = pss + jnp.sum(pf * pf, axis=0, keepdims=True)
            s, ss = ps, pss

    return body


def _group_mat(c, groups):
    gidx = jnp.arange(c) // (c // groups)
    return (gidx[:, None] == gidx[None, :]).astype(jnp.float32)


def kernel(x, conv_in_w, conv_in_b,
           r0_gn1_gamma, r0_gn1_beta, r0_conv1_w, r0_conv1_b,
           r0_gn2_gamma, r0_gn2_beta, r0_conv2_w, r0_conv2_b,
           r1_gn1_gamma, r1_gn1_beta, r1_conv1_w, r1_conv1_b,
           r1_gn2_gamma, r1_gn2_beta, r1_conv2_w, r1_conv2_b,
           r1_sc_w, r1_sc_b,
           r2_gn1_gamma, r2_gn1_beta, r2_conv1_w, r2_conv1_b,
           r2_gn2_gamma, r2_gn2_beta, r2_conv2_w, r2_conv2_b,
           r2_sc_w, r2_sc_b):
    groups = 32
    f32, bf16 = jnp.float32, jnp.bfloat16
    B, c0, hr, wr = x.shape
    H, W = hr // 2, wr // 2
    cu = c0 * 4
    xu = (x.reshape(B, c0, H, 2, W, 2).transpose(0, 2, 4, 1, 3, 5)
          .reshape(B, H * W, cu).astype(bf16))

    raw = [
        dict(gn1=(r0_gn1_gamma, r0_gn1_beta), w1=r0_conv1_w, b1=r0_conv1_b,
             gn2=(r0_gn2_gamma, r0_gn2_beta), w2=r0_conv2_w, b2=r0_conv2_b,
             sc=None),
        dict(gn1=(r1_gn1_gamma, r1_gn1_beta), w1=r1_conv1_w, b1=r1_conv1_b,
             gn2=(r1_gn2_gamma, r1_gn2_beta), w2=r1_conv2_w, b2=r1_conv2_b,
             sc=(r1_sc_w, r1_sc_b)),
        dict(gn1=(r2_gn1_gamma, r2_gn1_beta), w1=r2_conv1_w, b1=r2_conv1_b,
             gn2=(r2_gn2_gamma, r2_gn2_beta), w2=r2_conv2_w, b2=r2_conv2_b,
             sc=(r2_sc_w, r2_sc_b)),
    ]

    cin0 = conv_in_w.shape[1]
    meta = []
    args = []
    in_specs = []

    def _add(arr):
        shp = arr.shape
        in_specs.append(pl.BlockSpec(shp, lambda *_: (0,) * len(shp)))
        args.append(arr)

    S = H
    csets = []
    for bp in raw:
        cin, cout = bp["w1"].shape[2], bp["w1"].shape[3]
        for c in (cin, cout):
            if c not in csets:
                csets.append(c)
        meta.append((S, cin, cout, cin // groups, cout // groups,
                     bp["sc"] is not None, bp is not raw[-1]))
        S //= 2
    assert len(csets) == 3

    in_specs.append(pl.BlockSpec((1, H * W, cu), lambda b: (b, 0, 0)))
    args.append(xu)
    _add(conv_in_w.astype(bf16))
    _add(conv_in_b.reshape(1, cin0).astype(f32))
    for c in csets:
        _add(_group_mat(c, groups))
    for bp, (S, cin, cout, *_r) in zip(raw, meta):
        _add(bp["gn1"][0].reshape(1, cin).astype(f32))
        _add(bp["gn1"][1].reshape(1, cin).astype(f32))
        _add(bp["w1"].reshape(9 * cin, cout).astype(bf16))
        _add(bp["b1"].reshape(1, cout).astype(f32))
        _add(bp["gn2"][0].reshape(1, cout).astype(f32))
        _add(bp["gn2"][1].reshape(1, cout).astype(f32))
        _add(bp["w2"].reshape(9 * cout, cout).astype(bf16))
        _add(bp["b2"].reshape(1, cout).astype(f32))
        if bp["sc"] is not None:
            _add(bp["sc"][0].astype(bf16))
            _add(bp["sc"][1].reshape(1, cout).astype(f32))

    out_shapes = []
    out_specs = []
    for (S, cin, cout, *_r) in meta:
        out_shapes.append(jax.ShapeDtypeStruct((B, S * S, cout), f32))
        out_specs.append(pl.BlockSpec((1, S * S, cout), lambda b: (b, 0, 0)))

    (S0, ci0, co0, *_), (S1, ci1, co1, *_), (S2, ci2, co2, *_) = meta
    scratch = [pltpu.VMEM((S0 * S0, ci0), bf16),
               pltpu.VMEM((S0 * S0, co0), bf16),
               pltpu.VMEM((S1 * S1, ci1), bf16),
               pltpu.VMEM((S1 * S1, co1), bf16),
               pltpu.VMEM((S2 * S2, ci2), bf16),
               pltpu.VMEM((S2 * S2, co2), bf16)]

    feats = pl.pallas_call(
        _make_net_kernel(H, meta),
        out_shape=tuple(out_shapes),
        grid=(B,),
        in_specs=in_specs,
        out_specs=tuple(out_specs),
        scratch_shapes=scratch,
        compiler_params=pltpu.CompilerParams(
            dimension_semantics=("parallel",),
            vmem_limit_bytes=_VMEM_LIMIT),
    )(*args)

    return [f.reshape(B, Si * Si, co).reshape(B, Si, Si, co)
            .transpose(0, 3, 1, 2)
            for f, (Si, ci, co, *_r) in zip(feats, meta)]


# speedup vs baseline: 1.0581x; 1.0413x over previous
import jax
import jax.numpy as jnp
from jax import lax
from jax.experimental import pallas as pl
from jax.experimental.pallas import tpu as pltpu

_VMEM_LIMIT = 100 * 1024 * 1024
_EPS = 1e-6


def _scale_shift(s, ss, gm_ref, g_ref, bt_ref, inv):
    mean = jnp.dot(s, gm_ref[...], preferred_element_type=jnp.float32,
                   precision=lax.Precision.HIGHEST) * inv
    ex2 = jnp.dot(ss, gm_ref[...], preferred_element_type=jnp.float32,
                  precision=lax.Precision.HIGHEST) * inv
    var = ex2 - mean * mean
    scale = g_ref[...] * lax.rsqrt(var + _EPS)
    shift = bt_ref[...] - mean * scale
    return scale, shift


def _conv_chunks(load, scale, shift, w_ref, cb_ref, S, cin, rt):
    for r0 in range(0, S, rt):
        lo = max(r0 - 1, 0)
        hi = min(r0 + rt + 1, S)
        y = load(lo * S, hi * S) * scale + shift
        y = 0.5 * y * (1.0 + jnp.tanh(0.5 * y))
        yb = y.astype(jnp.bfloat16).reshape(hi - lo, S, cin)
        sl = jnp.pad(yb, ((1 - (r0 - lo), 1 - (hi - r0 - rt)),
                          (1, 1), (0, 0)))
        patches = jnp.concatenate(
            [sl[dy:dy + rt, dx:dx + S, :].reshape(rt * S, cin)
             for dy in range(3) for dx in range(3)], axis=-1)
        yield r0, (jnp.dot(patches, w_ref[...],
                           preferred_element_type=jnp.float32) + cb_ref[...])


def _make_net_kernel(S0, cin0, meta):
    def body(*refs):
        xu_ref, wst_ref, bst_ref, gm_a, gm_b, gm_c = refs[:6]
        gms = {}
        for r in (gm_a, gm_b, gm_c):
            gms[r.shape[0]] = r
        k = 6
        bparams = []
        for (S, cin, cout, cg1, cg2, has_proj, do_pool) in meta:
            nper = 8 + (2 if has_proj else 0)
            bparams.append(refs[k:k + nper])
            k += nper
        f_refs = refs[k:k + 3]
        x0_s, h0_s, p0_s, h1_s, p1_s, h2_s = refs[k + 3:k + 9]
        h_scr = [h0_s, h1_s, h2_s]
        in_scr = [x0_s, p0_s, p1_s]

        acc4 = jnp.dot(xu_ref[0], wst_ref[...],
                       preferred_element_type=jnp.float32) + bst_ref[...]
        x0_s[...] = acc4.reshape(S0 * S0, cin0).astype(x0_s.dtype)
        s4 = jnp.sum(acc4, axis=0, keepdims=True)
        ss4 = jnp.sum(acc4 * acc4, axis=0, keepdims=True)
        s = sum(s4[:, p * cin0:(p + 1) * cin0] for p in range(4))
        ss = sum(ss4[:, p * cin0:(p + 1) * cin0] for p in range(4))

        for i, (S, cin, cout, cg1, cg2, has_proj, do_pool) in enumerate(meta):
            prm = bparams[i]
            if has_proj:
                (g1, b1, w1, cb1, g2, b2, w2, cb2, scw, scb) = prm
            else:
                (g1, b1, w1, cb1, g2, b2, w2, cb2) = prm
                scw = scb = None
            x_s = in_scr[i]
            h_s = h_scr[i]
            rt = max(2, min(8, 512 // S, S))
            hw = S * S

            scale, shift = _scale_shift(s, ss, gms[cin], g1, b1,
                                        1.0 / float(hw * cg1))
            ts = tss = 0.0
            for r0, a1 in _conv_chunks(
                    lambda a, b: x_s[a:b, :].astype(jnp.float32),
                    scale, shift, w1, cb1, S, cin, rt):
                h_s[r0 * S:(r0 + rt) * S, :] = a1.astype(h_s.dtype)
                ts = ts + jnp.sum(a1, axis=0, keepdims=True)
                tss = tss + jnp.sum(a1 * a1, axis=0, keepdims=True)

            scale, shift = _scale_shift(ts, tss, gms[cout], g2, b2,
                                        1.0 / float(hw * cg2))
            ps = pss = 0.0
            for r0, a2 in _conv_chunks(
                    lambda a, b: h_s[a:b, :].astype(jnp.float32),
                    scale, shift, w2, cb2, S, cout, rt):
                a, b = r0 * S, (r0 + rt) * S
                if has_proj:
                    a2 = a2 + (jnp.dot(x_s[a:b, :], scw[...],
                                       preferred_element_type=jnp.float32)
                               + scb[...])
                else:
                    a2 = a2 + x_s[a:b, :].astype(jnp.float32)
                f_refs[i][0, a:b, :] = a2.astype(f_refs[i].dtype)
                if do_pool:
                    v = a2.reshape(rt // 2, 2, S // 2, 2, cout)
                    pq = 0.25 * (v[:, 0, :, 0, :] + v[:, 0, :, 1, :]
                                 + v[:, 1, :, 0, :] + v[:, 1, :, 1, :])
                    pf = pq.reshape(rt * S // 4, cout)
                    in_scr[i + 1][(r0 // 2) * (S // 2):
                                  (r0 // 2 + rt // 2) * (S // 2), :] = (
                        pf.astype(in_scr[i + 1].dtype))
                    ps = ps + jnp.sum(pf, axis=0, keepdims=True)
                    pss = pss + jnp.sum(pf * pf, axis=0, keepdims=True)
            s, ss = ps, pss

    return body


def _group_mat(c, groups):
    gidx = jnp.arange(c) // (c // groups)
    return (gidx[:, None] == gidx[None, :]).astype(jnp.float32)


def kernel(x, conv_in_w, conv_in_b,
           r0_gn1_gamma, r0_gn1_beta, r0_conv1_w, r0_conv1_b,
           r0_gn2_gamma, r0_gn2_beta, r0_conv2_w, r0_conv2_b,
           r1_gn1_gamma, r1_gn1_beta, r1_conv1_w, r1_conv1_b,
           r1_gn2_gamma, r1_gn2_beta, r1_conv2_w, r1_conv2_b,
           r1_sc_w, r1_sc_b,
           r2_gn1_gamma, r2_gn1_beta, r2_conv1_w, r2_conv1_b,
           r2_gn2_gamma, r2_gn2_beta, r2_conv2_w, r2_conv2_b,
           r2_sc_w, r2_sc_b):
    groups = 32
    f32, bf16 = jnp.float32, jnp.bfloat16
    B, c0, hr, wr = x.shape
    H, W = hr // 2, wr // 2
    cu = c0 * 4
    xu = (x.reshape(B, c0, H, 2, W, 2).transpose(0, 2, 4, 1, 3, 5)
          .reshape(B, H * W, cu).astype(bf16))

    raw = [
        dict(gn1=(r0_gn1_gamma, r0_gn1_beta), w1=r0_conv1_w, b1=r0_conv1_b,
             gn2=(r0_gn2_gamma, r0_gn2_beta), w2=r0_conv2_w, b2=r0_conv2_b,
             sc=None),
        dict(gn1=(r1_gn1_gamma, r1_gn1_beta), w1=r1_conv1_w, b1=r1_conv1_b,
             gn2=(r1_gn2_gamma, r1_gn2_beta), w2=r1_conv2_w, b2=r1_conv2_b,
             sc=(r1_sc_w, r1_sc_b)),
        dict(gn1=(r2_gn1_gamma, r2_gn1_beta), w1=r2_conv1_w, b1=r2_conv1_b,
             gn2=(r2_gn2_gamma, r2_gn2_beta), w2=r2_conv2_w, b2=r2_conv2_b,
             sc=(r2_sc_w, r2_sc_b)),
    ]

    cin0 = conv_in_w.shape[1]
    meta = []
    args = []
    in_specs = []

    def _add(arr):
        shp = arr.shape
        in_specs.append(pl.BlockSpec(shp, lambda *_: (0,) * len(shp)))
        args.append(arr)

    S = H
    csets = []
    for bp in raw:
        cin, cout = bp["w1"].shape[2], bp["w1"].shape[3]
        for c in (cin, cout):
            if c not in csets:
                csets.append(c)
        meta.append((S, cin, cout, cin // groups, cout // groups,
                     bp["sc"] is not None, bp is not raw[-1]))
        S //= 2
    assert len(csets) == 3

    in_specs.append(pl.BlockSpec((1, H * W // 4, 4 * cu), lambda b: (b, 0, 0)))
    args.append(xu.reshape(B, H * W // 4, 4 * cu))
    _add(jnp.kron(jnp.eye(4, dtype=f32), conv_in_w).astype(bf16))
    _add(jnp.tile(conv_in_b.reshape(1, cin0), (1, 4)).astype(f32))
    for c in csets:
        _add(_group_mat(c, groups))
    for bp, (S, cin, cout, *_r) in zip(raw, meta):
        _add(bp["gn1"][0].reshape(1, cin).astype(f32))
        _add(bp["gn1"][1].reshape(1, cin).astype(f32))
        _add(bp["w1"].reshape(9 * cin, cout).astype(bf16))
        _add(bp["b1"].reshape(1, cout).astype(f32))
        _add(bp["gn2"][0].reshape(1, cout).astype(f32))
        _add(bp["gn2"][1].reshape(1, cout).astype(f32))
        _add(bp["w2"].reshape(9 * cout, cout).astype(bf16))
        _add(bp["b2"].reshape(1, cout).astype(f32))
        if bp["sc"] is not None:
            _add(bp["sc"][0].astype(bf16))
            _add(bp["sc"][1].reshape(1, cout).astype(f32))

    out_shapes = []
    out_specs = []
    for (S, cin, cout, *_r) in meta:
        out_shapes.append(jax.ShapeDtypeStruct((B, S * S, cout), f32))
        out_specs.append(pl.BlockSpec((1, S * S, cout), lambda b: (b, 0, 0)))

    (S0, ci0, co0, *_), (S1, ci1, co1, *_), (S2, ci2, co2, *_) = meta
    scratch = [pltpu.VMEM((S0 * S0, ci0), bf16),
               pltpu.VMEM((S0 * S0, co0), bf16),
               pltpu.VMEM((S1 * S1, ci1), bf16),
               pltpu.VMEM((S1 * S1, co1), bf16),
               pltpu.VMEM((S2 * S2, ci2), bf16),
               pltpu.VMEM((S2 * S2, co2), bf16)]

    feats = pl.pallas_call(
        _make_net_kernel(H, cin0, meta),
        out_shape=tuple(out_shapes),
        grid=(B,),
        in_specs=in_specs,
        out_specs=tuple(out_specs),
        scratch_shapes=scratch,
        compiler_params=pltpu.CompilerParams(
            dimension_semantics=("parallel",),
            vmem_limit_bytes=_VMEM_LIMIT),
    )(*args)

    return [f.reshape(B, Si * Si, co).reshape(B, Si, Si, co)
            .transpose(0, 3, 1, 2)
            for f, (Si, ci, co, *_r) in zip(feats, meta)]


# speedup vs baseline: 1.0639x; 1.0055x over previous
import jax
import jax.numpy as jnp
from jax import lax
from jax.experimental import pallas as pl
from jax.experimental.pallas import tpu as pltpu

_VMEM_LIMIT = 100 * 1024 * 1024
_EPS = 1e-6


def _scale_shift(s, ss, gm_ref, g_ref, bt_ref, inv):
    mean = jnp.dot(s, gm_ref[...], preferred_element_type=jnp.float32,
                   precision=lax.Precision.HIGHEST) * inv
    ex2 = jnp.dot(ss, gm_ref[...], preferred_element_type=jnp.float32,
                  precision=lax.Precision.HIGHEST) * inv
    var = ex2 - mean * mean
    scale = g_ref[...] * lax.rsqrt(var + _EPS)
    shift = bt_ref[...] - mean * scale
    return scale, shift


def _conv_chunks(load, scale, shift, w_ref, cb_ref, S, cin, rt):
    for r0 in range(0, S, rt):
        lo = max(r0 - 1, 0)
        hi = min(r0 + rt + 1, S)
        y = load(lo * S, hi * S) * scale + shift
        y = 0.5 * y * (1.0 + jnp.tanh(0.5 * y))
        yb = y.astype(jnp.bfloat16).reshape(hi - lo, S, cin)
        sl = jnp.pad(yb, ((1 - (r0 - lo), 1 - (hi - r0 - rt)),
                          (1, 1), (0, 0)))
        patches = jnp.concatenate(
            [sl[dy:dy + rt, dx:dx + S, :].reshape(rt * S, cin)
             for dy in range(3) for dx in range(3)], axis=-1)
        yield r0, (jnp.dot(patches, w_ref[...],
                           preferred_element_type=jnp.float32) + cb_ref[...])


def _make_net_kernel(S0, cin0, meta):
    def body(*refs):
        xu_ref, wst_ref, bst_ref, gm_a, gm_b, gm_c = refs[:6]
        gms = {}
        for r in (gm_a, gm_b, gm_c):
            gms[r.shape[0]] = r
        k = 6
        bparams = []
        for (S, cin, cout, cg1, cg2, has_proj, do_pool) in meta:
            nper = 8 + (2 if has_proj else 0)
            bparams.append(refs[k:k + nper])
            k += nper
        f_refs = refs[k:k + 3]
        x0_s, h0_s, p0_s, h1_s, p1_s, h2_s = refs[k + 3:k + 9]
        h_scr = [h0_s, h1_s, h2_s]
        in_scr = [x0_s, p0_s, p1_s]

        acc4 = jnp.dot(xu_ref[0], wst_ref[...],
                       preferred_element_type=jnp.float32) + bst_ref[...]
        x0_s[...] = acc4.reshape(S0 * S0, cin0).astype(x0_s.dtype)
        s4 = jnp.sum(acc4, axis=0, keepdims=True)
        ss4 = jnp.sum(acc4 * acc4, axis=0, keepdims=True)
        s = sum(s4[:, p * cin0:(p + 1) * cin0] for p in range(4))
        ss = sum(ss4[:, p * cin0:(p + 1) * cin0] for p in range(4))

        for i, (S, cin, cout, cg1, cg2, has_proj, do_pool) in enumerate(meta):
            prm = bparams[i]
            if has_proj:
                (g1, b1, w1, cb1, g2, b2, w2, cb2, scw, scb) = prm
            else:
                (g1, b1, w1, cb1, g2, b2, w2, cb2) = prm
                scw = scb = None
            x_s = in_scr[i]
            h_s = h_scr[i]
            rt = max(2, min(16, 1024 // S, S))
            hw = S * S

            scale, shift = _scale_shift(s, ss, gms[cin], g1, b1,
                                        1.0 / float(hw * cg1))
            ts = tss = 0.0
            for r0, a1 in _conv_chunks(
                    lambda a, b: x_s[a:b, :].astype(jnp.float32),
                    scale, shift, w1, cb1, S, cin, rt):
                h_s[r0 * S:(r0 + rt) * S, :] = a1.astype(h_s.dtype)
                ts = ts + jnp.sum(a1, axis=0, keepdims=True)
                tss = tss + jnp.sum(a1 * a1, axis=0, keepdims=True)

            scale, shift = _scale_shift(ts, tss, gms[cout], g2, b2,
                                        1.0 / float(hw * cg2))
            ps = pss = 0.0
            for r0, a2 in _conv_chunks(
                    lambda a, b: h_s[a:b, :].astype(jnp.float32),
                    scale, shift, w2, cb2, S, cout, rt):
                a, b = r0 * S, (r0 + rt) * S
                if has_proj:
                    a2 = a2 + (jnp.dot(x_s[a:b, :], scw[...],
                                       preferred_element_type=jnp.float32)
                               + scb[...])
                else:
                    a2 = a2 + x_s[a:b, :].astype(jnp.float32)
                f_refs[i][0, a:b, :] = a2.astype(f_refs[i].dtype)
                if do_pool:
                    v = a2.reshape(rt // 2, 2, S // 2, 2, cout)
                    pq = 0.25 * (v[:, 0, :, 0, :] + v[:, 0, :, 1, :]
                                 + v[:, 1, :, 0, :] + v[:, 1, :, 1, :])
                    pf = pq.reshape(rt * S // 4, cout)
                    in_scr[i + 1][(r0 // 2) * (S // 2):
                                  (r0 // 2 + rt // 2) * (S // 2), :] = (
                        pf.astype(in_scr[i + 1].dtype))
                    ps = ps + jnp.sum(pf, axis=0, keepdims=True)
                    pss = pss + jnp.sum(pf * pf, axis=0, keepdims=True)
            s, ss = ps, pss

    return body


def _group_mat(c, groups):
    gidx = jnp.arange(c) // (c // groups)
    return (gidx[:, None] == gidx[None, :]).astype(jnp.float32)


def kernel(x, conv_in_w, conv_in_b,
           r0_gn1_gamma, r0_gn1_beta, r0_conv1_w, r0_conv1_b,
           r0_gn2_gamma, r0_gn2_beta, r0_conv2_w, r0_conv2_b,
           r1_gn1_gamma, r1_gn1_beta, r1_conv1_w, r1_conv1_b,
           r1_gn2_gamma, r1_gn2_beta, r1_conv2_w, r1_conv2_b,
           r1_sc_w, r1_sc_b,
           r2_gn1_gamma, r2_gn1_beta, r2_conv1_w, r2_conv1_b,
           r2_gn2_gamma, r2_gn2_beta, r2_conv2_w, r2_conv2_b,
           r2_sc_w, r2_sc_b):
    groups = 32
    f32, bf16 = jnp.float32, jnp.bfloat16
    B, c0, hr, wr = x.shape
    H, W = hr // 2, wr // 2
    cu = c0 * 4
    xu = (x.reshape(B, c0, H, 2, W, 2).transpose(0, 2, 4, 1, 3, 5)
          .reshape(B, H * W, cu).astype(bf16))

    raw = [
        dict(gn1=(r0_gn1_gamma, r0_gn1_beta), w1=r0_conv1_w, b1=r0_conv1_b,
             gn2=(r0_gn2_gamma, r0_gn2_beta), w2=r0_conv2_w, b2=r0_conv2_b,
             sc=None),
        dict(gn1=(r1_gn1_gamma, r1_gn1_beta), w1=r1_conv1_w, b1=r1_conv1_b,
             gn2=(r1_gn2_gamma, r1_gn2_beta), w2=r1_conv2_w, b2=r1_conv2_b,
             sc=(r1_sc_w, r1_sc_b)),
        dict(gn1=(r2_gn1_gamma, r2_gn1_beta), w1=r2_conv1_w, b1=r2_conv1_b,
             gn2=(r2_gn2_gamma, r2_gn2_beta), w2=r2_conv2_w, b2=r2_conv2_b,
             sc=(r2_sc_w, r2_sc_b)),
    ]

    cin0 = conv_in_w.shape[1]
    meta = []
    args = []
    in_specs = []

    def _add(arr):
        shp = arr.shape
        in_specs.append(pl.BlockSpec(shp, lambda *_: (0,) * len(shp)))
        args.append(arr)

    S = H
    csets = []
    for bp in raw:
        cin, cout = bp["w1"].shape[2], bp["w1"].shape[3]
        for c in (cin, cout):
            if c not in csets:
                csets.append(c)
        meta.append((S, cin, cout, cin // groups, cout // groups,
                     bp["sc"] is not None, bp is not raw[-1]))
        S //= 2
    assert len(csets) == 3

    in_specs.append(pl.BlockSpec((1, H * W // 4, 4 * cu), lambda b: (b, 0, 0)))
    args.append(xu.reshape(B, H * W // 4, 4 * cu))
    _add(jnp.kron(jnp.eye(4, dtype=f32), conv_in_w).astype(bf16))
    _add(jnp.tile(conv_in_b.reshape(1, cin0), (1, 4)).astype(f32))
    for c in csets:
        _add(_group_mat(c, groups))
    for bp, (S, cin, cout, *_r) in zip(raw, meta):
        _add(bp["gn1"][0].reshape(1, cin).astype(f32))
        _add(bp["gn1"][1].reshape(1, cin).astype(f32))
        _add(bp["w1"].reshape(9 * cin, cout).astype(bf16))
        _add(bp["b1"].reshape(1, cout).astype(f32))
        _add(bp["gn2"][0].reshape(1, cout).astype(f32))
        _add(bp["gn2"][1].reshape(1, cout).astype(f32))
        _add(bp["w2"].reshape(9 * cout, cout).astype(bf16))
        _add(bp["b2"].reshape(1, cout).astype(f32))
        if bp["sc"] is not None:
            _add(bp["sc"][0].astype(bf16))
            _add(bp["sc"][1].reshape(1, cout).astype(f32))

    out_shapes = []
    out_specs = []
    for (S, cin, cout, *_r) in meta:
        out_shapes.append(jax.ShapeDtypeStruct((B, S * S, cout), f32))
        out_specs.append(pl.BlockSpec((1, S * S, cout), lambda b: (b, 0, 0)))

    (S0, ci0, co0, *_), (S1, ci1, co1, *_), (S2, ci2, co2, *_) = meta
    scratch = [pltpu.VMEM((S0 * S0, ci0), bf16),
               pltpu.VMEM((S0 * S0, co0), bf16),
               pltpu.VMEM((S1 * S1, ci1), bf16),
               pltpu.VMEM((S1 * S1, co1), bf16),
               pltpu.VMEM((S2 * S2, ci2), bf16),
               pltpu.VMEM((S2 * S2, co2), bf16)]

    feats = pl.pallas_call(
        _make_net_kernel(H, cin0, meta),
        out_shape=tuple(out_shapes),
        grid=(B,),
        in_specs=in_specs,
        out_specs=tuple(out_specs),
        scratch_shapes=scratch,
        compiler_params=pltpu.CompilerParams(
            dimension_semantics=("parallel",),
            vmem_limit_bytes=_VMEM_LIMIT),
    )(*args)

    return [f.reshape(B, Si * Si, co).reshape(B, Si, Si, co)
            .transpose(0, 3, 1, 2)
            for f, (Si, ci, co, *_r) in zip(feats, meta)]
